# baseline (speedup 1.0000x reference)
import functools
import math

import jax
import jax.numpy as jnp
from jax import lax
from jax.experimental import pallas as pl
from jax.experimental.pallas import tpu as pltpu
from jax.experimental.pallas import tpu_sc as plsc

_D = 1024
_E = 16
_H = 4096
_T = 128
_CAP = float(_T)
_EPS = 1e-6
_HT = 2048
_NW = 32
_TPW = _T // _NW


def _logits_body(x_ref, wg_ref, bg_ref, out_ref):
    out_ref[...] = jnp.dot(x_ref[...], wg_ref[...],
                           preferred_element_type=jnp.float32) + bg_ref[...]


def _sc_group(lg_v, iota, base):
    vs = [plsc.load_gather(lg_v, [iota * _E + (base + e)]) for e in range(_E)]
    m = vs[0]
    for e in range(1, _E):
        m = jnp.maximum(m, vs[e])
    exs = [jnp.exp(vs[e] - m) for e in range(_E)]
    s = exs[0]
    for e in range(1, _E):
        s = s + exs[e]
    fi = jnp.full((16,), _E, jnp.int32)
    for e in range(_E - 1, -1, -1):
        fi = jnp.where(vs[e] == m, e, fi)
    return [jnp.where(fi == e, exs[e] / s, 0.0) for e in range(_E)]


def _gate_sc_body(lg_hbm, gate_hbm, lg_v, acc_v, mask_v, inv_v, out_v):
    wid = lax.axis_index("s") * 2 + lax.axis_index("c")
    pltpu.sync_copy(lg_hbm, lg_v)
    iota = lax.iota(jnp.int32, 16)

    acc = [jnp.zeros((16,), jnp.float32) for _ in range(_E)]
    for g in range(_T // 16):
        masked = _sc_group(lg_v, iota, g * 16 * _E)
        for e in range(_E):
            mask_v[pl.ds(g * 256 + e * 16, 16)] = masked[e]
        acc = [a + mvec for a, mvec in zip(acc, masked)]
    for e in range(_E):
        acc_v[pl.ds(e * 16, 16)] = acc[e]
    den = jnp.full((16,), _EPS, jnp.float32)
    for tp in range(16):
        den = den + plsc.load_gather(acc_v, [iota * 16 + tp])
    inv_v[pl.ds(16, 16)] = _CAP / den

    @pl.when(wid < _T // 16)
    def _emit():
        base = wid * 256
        for e in range(_E):
            mvec = mask_v[pl.ds(base + e * 16, 16)]
            scale = plsc.load_gather(
                inv_v, [jnp.full((16,), 16 + e, jnp.int32)])
            plsc.store_scatter(out_v, [iota * _E + e], mvec * scale)
        pltpu.sync_copy(out_v, gate_hbm.at[pl.ds(wid * (16 * _E), 16 * _E)])


_NPRE = 2


def _pre_body(x_ref, w1_ref, b1_ref, w2_ref, b2_ref, out_ref):
    j = pl.program_id(1)
    h = jnp.dot(x_ref[...], w1_ref[0],
                preferred_element_type=jnp.float32) + b1_ref[0]
    h = 0.5 * h * (1.0 + lax.erf(h * (1.0 / math.sqrt(2.0))))
    part = jnp.dot(h, w2_ref[0], preferred_element_type=jnp.float32)

    @pl.when(j == 0)
    def _():
        out_ref[0] = part + b2_ref[0]

    @pl.when(j != 0)
    def _():
        out_ref[0] += part


def _ffn_body(x_ref, gate_ref, pre_ref, w1_ref, b1_ref, w2_ref, b2_ref,
              out_ref):
    e = pl.program_id(0)
    j = pl.program_id(1)
    iota = lax.broadcasted_iota(jnp.int32, (_T, _E), 1)

    @pl.when((e == 0) & (j == 0))
    def _init():
        acc = jnp.zeros((_T, _D), jnp.float32)
        for pe in range(_NPRE):
            gp = jnp.sum(jnp.where(iota == pe, gate_ref[...], 0.0),
                         axis=1, keepdims=True)
            acc = acc + gp * pre_ref[pe]
        out_ref[...] = acc

    g = jnp.sum(jnp.where(iota == (e + _NPRE), gate_ref[...], 0.0),
                axis=1, keepdims=True)
    h = jnp.dot(x_ref[...], w1_ref[0],
                preferred_element_type=jnp.float32) + b1_ref[0]
    h = 0.5 * h * (1.0 + lax.erf(h * (1.0 / math.sqrt(2.0))))
    out_ref[...] += jnp.dot(g * h, w2_ref[0],
                            preferred_element_type=jnp.float32)

    @pl.when(j == 0)
    def _bias2():
        out_ref[...] += g * b2_ref[0]


def kernel(x, Wg, bg, W1, b1, W2, b2):
    logits = pl.pallas_call(
        _logits_body,
        out_shape=jax.ShapeDtypeStruct((_T, _E), jnp.float32),
    )(x, Wg, bg.reshape(1, _E))

    sc_mesh = plsc.VectorSubcoreMesh(core_axis_name="c", subcore_axis_name="s")
    gate1d = pl.kernel(
        _gate_sc_body,
        mesh=sc_mesh,
        out_type=jax.ShapeDtypeStruct((_T * _E,), jnp.float32),
        scratch_types=[
            pltpu.VMEM((_T * _E,), jnp.float32),
            pltpu.VMEM((_E * 16,), jnp.float32),
            pltpu.VMEM((_T * _E,), jnp.float32),
            pltpu.VMEM((32,), jnp.float32),
            pltpu.VMEM((16 * _E,), jnp.float32),
        ],
        compiler_params=pltpu.CompilerParams(needs_layout_passes=False),
    )(logits.reshape(_T * _E))
    gate = gate1d.reshape(_T, _E)

    nj = _H // _HT
    b1r = b1.reshape(_E, 1, _H)
    b2r = b2.reshape(_E, 1, _D)

    pre = pl.pallas_call(
        _pre_body,
        grid=(_NPRE, nj),
        in_specs=[
            pl.BlockSpec((_T, _D), lambda e, j: (0, 0)),
            pl.BlockSpec((1, _D, _HT), lambda e, j: (e, 0, j)),
            pl.BlockSpec((1, 1, _HT), lambda e, j: (e, 0, j)),
            pl.BlockSpec((1, _HT, _D), lambda e, j: (e, j, 0)),
            pl.BlockSpec((1, 1, _D), lambda e, j: (e, 0, 0)),
        ],
        out_specs=pl.BlockSpec((1, _T, _D), lambda e, j: (e, 0, 0)),
        out_shape=jax.ShapeDtypeStruct((_NPRE, _T, _D), jnp.float32),
        compiler_params=pltpu.CompilerParams(
            dimension_semantics=("arbitrary", "arbitrary"),
        ),
    )(x, W1, b1r, W2, b2r)

    out = pl.pallas_call(
        _ffn_body,
        grid=(_E - _NPRE, nj),
        in_specs=[
            pl.BlockSpec((_T, _D), lambda e, j: (0, 0)),
            pl.BlockSpec((_T, _E), lambda e, j: (0, 0)),
            pl.BlockSpec((_NPRE, _T, _D), lambda e, j: (0, 0, 0)),
            pl.BlockSpec((1, _D, _HT), lambda e, j: (e + _NPRE, 0, j)),
            pl.BlockSpec((1, 1, _HT), lambda e, j: (e + _NPRE, 0, j)),
            pl.BlockSpec((1, _HT, _D), lambda e, j: (e + _NPRE, j, 0)),
            pl.BlockSpec((1, 1, _D), lambda e, j: (e + _NPRE, 0, 0)),
        ],
        out_specs=pl.BlockSpec((_T, _D), lambda e, j: (0, 0)),
        out_shape=jax.ShapeDtypeStruct((_T, _D), jnp.float32),
        compiler_params=pltpu.CompilerParams(
            dimension_semantics=("arbitrary", "arbitrary"),
        ),
    )(x, gate, pre, W1, b1r, W2, b2r)
    return out


# speedup vs baseline: 1.1859x; 1.1859x over previous
import functools
import math

import jax
import jax.numpy as jnp
from jax import lax
from jax.experimental import pallas as pl
from jax.experimental.pallas import tpu as pltpu

_D = 1024
_E = 16
_H = 4096
_T = 128
_CAP = float(_T)
_EPS = 1e-6
_HT = 2048


def _ffn_body(x_ref, wg_ref, bg_ref, w1_ref, b1_ref, w2_ref, b2_ref,
              out_ref, gate_ref):
    e = pl.program_id(0)
    j = pl.program_id(1)

    @pl.when((e == 0) & (j == 0))
    def _gate_and_init():
        logits = jnp.dot(x_ref[...], wg_ref[...],
                         preferred_element_type=jnp.float32) + bg_ref[...]
        m = jnp.max(logits, axis=1, keepdims=True)
        ex = jnp.exp(logits - m)
        p = ex / jnp.sum(ex, axis=1, keepdims=True)
        iota = lax.broadcasted_iota(jnp.int32, (_T, _E), 1)
        pm = jnp.max(p, axis=1, keepdims=True)
        first = jnp.min(jnp.where(p >= pm, iota, _E), axis=1, keepdims=True)
        masked = jnp.where(iota == first, p, 0.0)
        denom = jnp.sum(masked, axis=0, keepdims=True) + _EPS
        gate_ref[...] = masked / denom * _CAP
        out_ref[...] = jnp.zeros_like(out_ref)

    iota = lax.broadcasted_iota(jnp.int32, (_T, _E), 1)
    g = jnp.sum(jnp.where(iota == e, gate_ref[...], 0.0),
                axis=1, keepdims=True)
    eiota1 = lax.broadcasted_iota(jnp.int32, (_E, _HT), 0)
    b1row = jnp.sum(jnp.where(eiota1 == e, b1_ref[:, pl.ds(j * _HT, _HT)],
                              0.0), axis=0, keepdims=True)
    h = jnp.dot(x_ref[...], w1_ref[0],
                preferred_element_type=jnp.float32) + b1row
    h = 0.5 * h * (1.0 + lax.erf(h * (1.0 / math.sqrt(2.0))))
    out_ref[...] += jnp.dot(g * h, w2_ref[0],
                            preferred_element_type=jnp.float32)

    @pl.when(j == 0)
    def _bias2():
        eiota2 = lax.broadcasted_iota(jnp.int32, (_E, _D), 0)
        b2row = jnp.sum(jnp.where(eiota2 == e, b2_ref[...], 0.0),
                        axis=0, keepdims=True)
        out_ref[...] += g * b2row


def kernel(x, Wg, bg, W1, b1, W2, b2):
    nj = _H // _HT
    out = pl.pallas_call(
        _ffn_body,
        grid=(_E, nj),
        in_specs=[
            pl.BlockSpec((_T, _D), lambda e, j: (0, 0)),
            pl.BlockSpec((_D, _E), lambda e, j: (0, 0)),
            pl.BlockSpec((_E,), lambda e, j: (0,)),
            pl.BlockSpec((1, _D, _HT), lambda e, j: (e, 0, j)),
            pl.BlockSpec((_E, _H), lambda e, j: (0, 0)),
            pl.BlockSpec((1, _HT, _D), lambda e, j: (e, j, 0)),
            pl.BlockSpec((_E, _D), lambda e, j: (0, 0)),
        ],
        out_specs=pl.BlockSpec((_T, _D), lambda e, j: (0, 0)),
        out_shape=jax.ShapeDtypeStruct((_T, _D), jnp.float32),
        scratch_shapes=[pltpu.VMEM((_T, _E), jnp.float32)],
        compiler_params=pltpu.CompilerParams(
            dimension_semantics=("arbitrary", "arbitrary"),
        ),
    )(x, Wg, bg, W1, b1, W2, b2)
    return out
